# Initial kernel scaffold; baseline (speedup 1.0000x reference)
import jax
import jax.numpy as jnp
from jax.experimental import pallas as pl


def kernel(edge_src, edge_dst, edge_rid, subj, rel, obj, ent_embeds, rel_embeds, lin_w, lin_b, gru_wih, gru_whh, gru_bih, gru_bhh, rgcn_wmsg, rgcn_wself, rgcn_b, obj_conv_w, obj_conv_b, obj_fc_w, obj_fc_b, rel_conv_w, rel_conv_b, rel_fc_w, rel_fc_b, lin2_w, lin2_b, mha_inw, mha_inb, mha_outw, mha_outb):
    raise NotImplementedError("write your pallas kernel here")


# speedup vs baseline: 3.0330x; 3.0330x over previous
import functools

import jax
import jax.numpy as jnp
from jax import lax
from jax.experimental import pallas as pl
from jax.experimental.pallas import tpu as pltpu
from jax.experimental.pallas import tpu_sc as plsc

N = 10000; R = 200; H = 128; T = 3; E = 160000; B = 1024
TOPK = 50; C = 50; KS = 3; NH = 4; L = 2
DH = H // NH

NC, NS = 2, 16
NW = NC * NS

TAB = R * N
TABP = TAB + 128
CNT_CH = 160
SEG_CH = 40
GK_CH = 13
GK_PAD = GK_CH * 128

_mesh = plsc.VectorSubcoreMesh(core_axis_name="c", subcore_axis_name="s")
_f32 = jnp.float32


def _zero_vmem_2d(ref, rows):
    z16 = jnp.zeros((16,), _f32)
    for i in range(rows):
        for j in range(ref.shape[1] // 16):
            ref[i, pl.ds(j * 16, 16)] = z16


def _zero_vmem_1d(ref):
    z16 = jnp.zeros((16,), _f32)
    for i in range(ref.shape[0] // 16):
        ref[pl.ds(i * 16, 16)] = z16


CNT_REM = TAB - 976 * 2048


@functools.partial(
    pl.kernel,
    out_type=(jax.ShapeDtypeStruct((TAB,), _f32),
              jax.ShapeDtypeStruct((TAB,), _f32)),
    mesh=_mesh,
    scratch_types=[
        pltpu.VMEM_SHARED((TABP,), _f32),
        pltpu.VMEM((16, 128), jnp.int32),
        pltpu.VMEM((128,), _f32),
        pltpu.VMEM((2048,), _f32),
    ],
)
def _sc_counts(idx_hbm, outa_hbm, outb_hbm, tab, idx_v, ones_v, zer_v):
    c = lax.axis_index("c")
    s = lax.axis_index("s")
    for i in range(8):
        ones_v[pl.ds(i * 16, 16)] = jnp.ones((16,), _f32)
    _zero_vmem_1d(zer_v)
    for k in range(61):
        pltpu.sync_copy(zer_v, tab.at[pl.ds((s + NS * k) * 2048, 2048)])

    @pl.when(s == 0)
    def _():
        pltpu.sync_copy(zer_v.at[pl.ds(0, CNT_REM)],
                        tab.at[pl.ds(976 * 2048, CNT_REM)])

    plsc.subcore_barrier()
    for g in range(CNT_CH // 16):
        pltpu.sync_copy(idx_hbm.at[c, s, pl.ds(g * 16, 16)], idx_v)
        for j in range(16):
            pltpu.sync_copy(ones_v, tab.at[idx_v.at[j]], add=True)
    plsc.subcore_barrier()

    def wb(dst):
        for k in range(61):
            off = (s + NS * k) * 2048
            pltpu.sync_copy(tab.at[pl.ds(off, 2048)], dst.at[pl.ds(off, 2048)])

        @pl.when(s == 0)
        def _():
            pltpu.sync_copy(tab.at[pl.ds(976 * 2048, CNT_REM)],
                            dst.at[pl.ds(976 * 2048, CNT_REM)])

    @pl.when(c == 0)
    def _():
        wb(outa_hbm)

    @pl.when(c == 1)
    def _():
        wb(outb_hbm)


@functools.partial(
    pl.kernel,
    out_type=jax.ShapeDtypeStruct((2, N, H), _f32),
    mesh=_mesh,
    scratch_types=[
        pltpu.VMEM_SHARED((N + 16, H), _f32),
        pltpu.VMEM((SEG_CH, 128), jnp.int32),
        pltpu.VMEM((SEG_CH, 128), jnp.int32),
        pltpu.VMEM((128, H), _f32),
        pltpu.VMEM((16, H), _f32),
        pltpu.SemaphoreType.DMA,
    ],
)
def _sc_segsum(h_hbm, src_hbm, dst_hbm, out_hbm, acc, src_v, dst_v, rows_v,
               zer_v, sem):
    c = lax.axis_index("c")
    s = lax.axis_index("s")
    wid = s * NC + c
    pltpu.sync_copy(src_hbm.at[wid], src_v)
    pltpu.sync_copy(dst_hbm.at[wid], dst_v)
    _zero_vmem_2d(zer_v, 16)
    for k in range(39):
        pltpu.sync_copy(zer_v, acc.at[pl.ds(s * 626 + k * 16, 16), :])
    pltpu.sync_copy(zer_v.at[pl.ds(0, 2), :], acc.at[pl.ds(s * 626 + 624, 2), :])
    plsc.subcore_barrier()
    for j in range(SEG_CH):
        pltpu.async_copy(h_hbm.at[src_v.at[j]], rows_v, sem).wait()
        pltpu.sync_copy(rows_v, acc.at[dst_v.at[j]], add=True)
    plsc.subcore_barrier()
    pltpu.sync_copy(acc.at[pl.ds(s * 624, 624), :],
                    out_hbm.at[c, pl.ds(s * 624, 624), :])

    @pl.when(s == NS - 1)
    def _():
        pltpu.sync_copy(acc.at[pl.ds(9984, 16), :],
                        out_hbm.at[c, pl.ds(9984, 16), :])


@functools.partial(
    pl.kernel,
    out_type=jax.ShapeDtypeStruct((3, B, H), _f32),
    mesh=_mesh,
    scratch_types=[
        pltpu.VMEM((3, 32), jnp.int32),
        pltpu.VMEM((32, H), _f32),
        pltpu.SemaphoreType.DMA,
    ],
)
def _sc_gather3(ent_hbm, rel_hbm, idx_hbm, out_hbm, idx_v, rows_v, sem):
    c = lax.axis_index("c")
    s = lax.axis_index("s")
    wid = s * NC + c
    pltpu.sync_copy(idx_hbm.at[wid], idx_v)
    for k, tbl in ((0, ent_hbm), (1, rel_hbm), (2, ent_hbm)):
        pltpu.async_copy(tbl.at[idx_v.at[k]], rows_v, sem).wait()
        pltpu.sync_copy(rows_v, out_hbm.at[k, pl.ds(wid * 32, 32), :])


@functools.partial(
    pl.kernel,
    out_type=jax.ShapeDtypeStruct((NW * GK_PAD, H), _f32),
    mesh=_mesh,
    scratch_types=[
        pltpu.VMEM((GK_CH, 128), jnp.int32),
        pltpu.VMEM((128, H), _f32),
        pltpu.SemaphoreType.DMA,
    ],
)
def _sc_gather_topk(ent_hbm, idx_hbm, out_hbm, idx_v, rows_v, sem):
    c = lax.axis_index("c")
    s = lax.axis_index("s")
    wid = s * NC + c
    pltpu.sync_copy(idx_hbm.at[wid], idx_v)
    for j in range(GK_CH):
        pltpu.async_copy(ent_hbm.at[idx_v.at[j]], rows_v, sem).wait()
        pltpu.sync_copy(rows_v, out_hbm.at[pl.ds(wid * GK_PAD + j * 128, 128), :])


def _dot(a, b):
    return lax.dot_general(a, b, (((1,), (0,)), ((), ())),
                           preferred_element_type=_f32)


def _dott(a, b):
    return lax.dot_general(a, b, (((1,), (1,)), ((), ())),
                           preferred_element_type=_f32)


def _l2n(x):
    n = jnp.sqrt(jnp.sum(x * x, axis=1, keepdims=True))
    return x / jnp.maximum(n, 1e-12)


def _l2norm_body(x_ref, o_ref):
    o_ref[...] = _l2n(x_ref[...])


def _l2norm(x, blk):
    m = x.shape[0]
    return pl.pallas_call(
        _l2norm_body,
        grid=(m // blk,),
        in_specs=[pl.BlockSpec((blk, H), lambda i: (i, 0))],
        out_specs=pl.BlockSpec((blk, H), lambda i: (i, 0)),
        out_shape=jax.ShapeDtypeStruct((m, H), _f32),
    )(x)


NP = 10240
NCH = 5
NBLK = NP // NCH


def _rel_body(cntA_ref, ent_ref, rel_emb_ref, rel_e_ref, wih_ref, whh_ref,
              bih_ref, bhh_ref, nrel_ref):
    m = (cntA_ref[...] > 0).astype(_f32)
    s = _dot(m, ent_ref[...])
    cnt = jnp.sum(m, axis=1, keepdims=True)
    rel_ent = jnp.where(cnt > 0, s / cnt, 0.0)
    x = jnp.concatenate([rel_ent, rel_emb_ref[...]], axis=1)
    h = rel_e_ref[...]
    gi = _dott(x, wih_ref[...]) + bih_ref[...]
    gh = _dott(h, whh_ref[...]) + bhh_ref[...]
    r = jax.nn.sigmoid(gi[:, :H] + gh[:, :H])
    z = jax.nn.sigmoid(gi[:, H:2 * H] + gh[:, H:2 * H])
    n = jnp.tanh(gi[:, 2 * H:] + r * gh[:, 2 * H:])
    nrel_ref[...] = _l2n((1.0 - z) * n + z * h)


def _rel_update(cntA, ent_e, rel_embeds, rel_e, wih, whh, bih, bhh):
    return pl.pallas_call(
        _rel_body,
        out_shape=jax.ShapeDtypeStruct((R, H), _f32),
    )(cntA, ent_e, rel_embeds, rel_e, wih, whh, bih, bhh)


LBLK = 1000


def _layer1_body(cnt_ref, gp_ref, nrel_ref, h_ref, wmsg_ref, wself_ref, b_ref,
                 h1_ref, F_ref):
    Fv = _dot(cnt_ref[...], nrel_ref[...])
    deg = jnp.maximum(jnp.sum(cnt_ref[...], axis=1, keepdims=True), 1.0)
    G = gp_ref[0] + gp_ref[1]
    agg = _dot(G + Fv, wmsg_ref[...]) / deg
    h1_ref[...] = jax.nn.relu(agg + _dot(h_ref[...], wself_ref[...]) +
                              b_ref[...])
    F_ref[...] = Fv


def _layer2_body(cnt_ref, gp_ref, F_ref, h_ref, wmsg_ref, wself_ref, b_ref,
                 ent_ref, linw_ref, linb_ref, out_ref):
    deg = jnp.maximum(jnp.sum(cnt_ref[...], axis=1, keepdims=True), 1.0)
    G = gp_ref[0] + gp_ref[1]
    agg = _dot(G + F_ref[...], wmsg_ref[...]) / deg
    hh = jax.nn.relu(agg + _dot(h_ref[...], wself_ref[...]) + b_ref[...])
    hn = _l2n(hh)
    e = ent_ref[...]
    u = jax.nn.sigmoid(_dott(e, linw_ref[...]) + linb_ref[...])
    out_ref[...] = e + u * (hn - e)


def _rgcn_layer1(cntB, gp, nrel, h, wmsg, wself, b):
    blk = lambda r, c, f=None: pl.BlockSpec((r, c), f or (lambda i: (0, 0)))
    return pl.pallas_call(
        _layer1_body,
        grid=(N // LBLK,),
        in_specs=[
            pl.BlockSpec((LBLK, R), lambda i: (i, 0)),
            pl.BlockSpec((2, LBLK, H), lambda i: (0, i, 0)),
            blk(R, H),
            pl.BlockSpec((LBLK, H), lambda i: (i, 0)),
            blk(H, H), blk(H, H), blk(1, H),
        ],
        out_specs=[pl.BlockSpec((LBLK, H), lambda i: (i, 0)),
                   pl.BlockSpec((LBLK, H), lambda i: (i, 0))],
        out_shape=[jax.ShapeDtypeStruct((N, H), _f32),
                   jax.ShapeDtypeStruct((N, H), _f32)],
    )(cntB, gp, nrel, h, wmsg, wself, b)


def _rgcn_layer2(cntB, gp, F, h, wmsg, wself, b, ent_e, lin_w, lin_b):
    blk = lambda r, c: pl.BlockSpec((r, c), lambda i: (0, 0))
    return pl.pallas_call(
        _layer2_body,
        grid=(N // LBLK,),
        in_specs=[
            pl.BlockSpec((LBLK, R), lambda i: (i, 0)),
            pl.BlockSpec((2, LBLK, H), lambda i: (0, i, 0)),
            pl.BlockSpec((LBLK, H), lambda i: (i, 0)),
            pl.BlockSpec((LBLK, H), lambda i: (i, 0)),
            blk(H, H), blk(H, H), blk(1, H),
            pl.BlockSpec((LBLK, H), lambda i: (i, 0)),
            blk(H, H), blk(1, H),
        ],
        out_specs=pl.BlockSpec((LBLK, H), lambda i: (i, 0)),
        out_shape=jax.ShapeDtypeStruct((N, H), _f32),
    )(cntB, gp, F, h, wmsg, wself, b, ent_e, lin_w, lin_b)


def _conv_transe_q(a, bb, wf_ref, cb_ref, fcp_ref, fcb_ref, rows):
    xs = (
        jnp.concatenate([jnp.zeros((rows, 1), _f32), a[:, :-1]], axis=1),
        a,
        jnp.concatenate([a[:, 1:], jnp.zeros((rows, 1), _f32)], axis=1),
        jnp.concatenate([jnp.zeros((rows, 1), _f32), bb[:, :-1]], axis=1),
        bb,
        jnp.concatenate([bb[:, 1:], jnp.zeros((rows, 1), _f32)], axis=1),
    )
    y = jnp.zeros((rows, H, C), _f32)
    for t in range(6):
        y = y + xs[t][:, :, None] * wf_ref[t:t + 1, :][None, :, :]
    y = jax.nn.relu(y + cb_ref[...][None])
    return jax.nn.relu(_dott(y.reshape(rows, H * C), fcp_ref[...]) +
                       fcb_ref[...])


OBLK = 128


def _objlogit_body(es_ref, er_ref, wf_ref, cb_ref, fcp_ref, fcb_ref, ent_ref,
                   out_ref, q_s):
    j = pl.program_id(1)

    @pl.when(j == 0)
    def _():
        q_s[...] = _conv_transe_q(es_ref[...], er_ref[...], wf_ref, cb_ref,
                                  fcp_ref, fcb_ref, OBLK)

    col = j * NBLK + lax.broadcasted_iota(jnp.int32, (1, NBLK), 1)
    out_ref[...] = jnp.where(col < N, _dott(q_s[...], ent_ref[...]),
                             -jnp.inf)


def _obj_logits(es, er, wf, cb, fcp, fcb, ent_e):
    blk = lambda r, c: pl.BlockSpec((r, c), lambda i, j: (0, 0))
    return pl.pallas_call(
        _objlogit_body,
        grid=(B // OBLK, NCH),
        in_specs=[
            pl.BlockSpec((OBLK, H), lambda i, j: (i, 0)),
            pl.BlockSpec((OBLK, H), lambda i, j: (i, 0)),
            blk(6, C), blk(1, C), blk(H, H * C), blk(1, H),
            pl.BlockSpec((NBLK, H), lambda i, j: (j, 0)),
        ],
        out_specs=pl.BlockSpec((OBLK, NBLK), lambda i, j: (i, j)),
        out_shape=jax.ShapeDtypeStruct((B, NP), _f32),
        scratch_shapes=[pltpu.VMEM((OBLK, H), _f32)],
    )(es, er, wf, cb, fcp, fcb, ent_e)


def _topk_body(lg_ref, oi_ref, x_s):
    x_s[...] = lg_ref[...]
    cols = lax.broadcasted_iota(jnp.int32, (OBLK, NP), 1)
    colsk = lax.broadcasted_iota(jnp.int32, (OBLK, TOPK), 1)

    def it(k, acc):
        v = x_s[...]
        m = jnp.max(v, axis=1, keepdims=True)
        am = jnp.min(jnp.where(v >= m, cols, NP), axis=1, keepdims=True)
        x_s[...] = jnp.where(cols == am, -jnp.inf, v)
        return jnp.where(colsk == k, am, acc)

    oi_ref[...] = lax.fori_loop(0, TOPK, it,
                                jnp.zeros((OBLK, TOPK), jnp.int32))


def _topk(logits):
    return pl.pallas_call(
        _topk_body,
        grid=(B // OBLK,),
        in_specs=[pl.BlockSpec((OBLK, NP), lambda i: (i, 0))],
        out_specs=pl.BlockSpec((OBLK, TOPK), lambda i: (i, 0)),
        out_shape=jax.ShapeDtypeStruct((B, TOPK), jnp.int32),
        scratch_shapes=[pltpu.VMEM((OBLK, NP), _f32)],
    )(logits)


MBLK = 128


def _mha_body(es_ref, er_ref, pred_ref, l2w_ref, l2b_ref, inw_ref, inb_ref,
              outw_ref, outb_ref, ent_ref, out_ref, rr_s):
    @pl.when(pl.program_id(1) == 0)
    def _():
        x2 = jnp.concatenate([es_ref[...], er_ref[...]], axis=1)
        q2 = _dott(x2, l2w_ref[...]) + l2b_ref[...]
        kv = pred_ref[...]
        Q = _dott(q2, inw_ref[0:H]) + inb_ref[:, 0:H]
        Kk = _dott(kv, inw_ref[H:2 * H]) + inb_ref[:, H:2 * H]
        V = _dott(kv, inw_ref[2 * H:]) + inb_ref[:, 2 * H:]
        Q4 = Q.reshape(MBLK, 1, NH, DH)
        K4 = Kk.reshape(MBLK, TOPK, NH, DH)
        V4 = V.reshape(MBLK, TOPK, NH, DH)
        sc = jnp.sum(Q4 * K4, axis=3) / jnp.sqrt(jnp.float32(DH))
        sc = sc - jnp.max(sc, axis=1, keepdims=True)
        ex = jnp.exp(sc)
        att = ex / jnp.sum(ex, axis=1, keepdims=True)
        o = jnp.sum(att[:, :, :, None] * V4, axis=1).reshape(MBLK, H)
        rr_s[...] = _dott(o, outw_ref[...]) + outb_ref[...]

    out_ref[...] = _dott(rr_s[...], ent_ref[...])


def _mha_logits(es, er, pred, l2w, l2b, inw, inb, outw, outb, ent_e):
    blk = lambda r, c: pl.BlockSpec((r, c), lambda i, j: (0, 0))
    return pl.pallas_call(
        _mha_body,
        grid=(B // MBLK, NCH),
        in_specs=[
            pl.BlockSpec((MBLK, H), lambda i, j: (i, 0)),
            pl.BlockSpec((MBLK, H), lambda i, j: (i, 0)),
            pl.BlockSpec((MBLK * TOPK, H), lambda i, j: (i, 0)),
            blk(H, 2 * H), blk(1, H), blk(3 * H, H), blk(1, 3 * H),
            blk(H, H), blk(1, H),
            pl.BlockSpec((NBLK, H), lambda i, j: (j, 0)),
        ],
        out_specs=pl.BlockSpec((MBLK, NBLK), lambda i, j: (i, j)),
        out_shape=jax.ShapeDtypeStruct((B, N), _f32),
        scratch_shapes=[pltpu.VMEM((MBLK, H), _f32)],
    )(es, er, pred, l2w, l2b, inw, inb, outw, outb, ent_e)


def _rellogit_body(es_ref, eo_ref, wf_ref, cb_ref, fcp_ref, fcb_ref, rel_ref,
                   out_ref):
    q = _conv_transe_q(es_ref[...], eo_ref[...], wf_ref, cb_ref, fcp_ref,
                       fcb_ref, OBLK)
    out_ref[...] = _dott(q, rel_ref[...])


def _rel_logits(es, eo, wf, cb, fcp, fcb, rel_e):
    blk = lambda r, c: pl.BlockSpec((r, c), lambda i: (0, 0))
    return pl.pallas_call(
        _rellogit_body,
        grid=(B // OBLK,),
        in_specs=[
            pl.BlockSpec((OBLK, H), lambda i: (i, 0)),
            pl.BlockSpec((OBLK, H), lambda i: (i, 0)),
            blk(6, C), blk(1, C), blk(H, H * C), blk(1, H),
            blk(R, H),
        ],
        out_specs=pl.BlockSpec((OBLK, R), lambda i: (i, 0)),
        out_shape=jax.ShapeDtypeStruct((B, R), _f32),
    )(es, eo, wf, cb, fcp, fcb, rel_e)


def _pad_chunks(flat, per_tile, nchunks, pad_val):
    x = flat.reshape(-1, per_tile)
    pad = nchunks * 128 - per_tile
    x = jnp.pad(x, ((0, 0), (0, pad)), constant_values=pad_val)
    return x.reshape(-1, nchunks, 128).astype(jnp.int32)


def kernel(edge_src, edge_dst, edge_rid, subj, rel, obj, ent_embeds,
           rel_embeds, lin_w, lin_b, gru_wih, gru_whh, gru_bih, gru_bhh,
           rgcn_wmsg, rgcn_wself, rgcn_b, obj_conv_w, obj_conv_b, obj_fc_w,
           obj_fc_b, rel_conv_w, rel_conv_b, rel_fc_w, rel_fc_b, lin2_w,
           lin2_b, mha_inw, mha_inb, mha_outw, mha_outb):
    f32 = _f32
    edge_src = edge_src.astype(jnp.int32)
    edge_dst = edge_dst.astype(jnp.int32)
    edge_rid = edge_rid.astype(jnp.int32)

    row2 = lambda v: v.reshape(1, -1).astype(f32)
    wf_obj = obj_conv_w.transpose(1, 2, 0).reshape(6, C).astype(f32)
    fcp_obj = obj_fc_w.reshape(H, C, H).transpose(0, 2, 1).reshape(H, H * C)
    wf_rel = rel_conv_w.transpose(1, 2, 0).reshape(6, C).astype(f32)
    fcp_rel = rel_fc_w.reshape(H, C, H).transpose(0, 2, 1).reshape(H, H * C)

    ent_e = _l2norm(ent_embeds.astype(f32), NBLK)
    rel_e = _l2norm(rel_embeds.astype(f32), R)

    for t in range(T):
        src, dst, rid = edge_src[t], edge_dst[t], edge_rid[t]
        idxA = jnp.concatenate([rid * N + src, rid * N + dst])
        idxA = _pad_chunks(idxA, 2 * E // NS, CNT_CH, TAB)
        idxB = _pad_chunks(dst * R + rid, E // NS, CNT_CH, TAB)
        cnta, cntb = _sc_counts(jnp.stack([idxA, idxB]))
        cntA = cnta.reshape(R, N)
        cntB = cntb.reshape(N, R)

        n_rel = _rel_update(cntA, ent_e, rel_embeds.astype(f32), rel_e,
                            gru_wih, gru_whh, row2(gru_bih), row2(gru_bhh))

        srcc = _pad_chunks(src, E // NW, SEG_CH, 0)
        dstc = _pad_chunks(dst, E // NW, SEG_CH, N)

        gp1 = _sc_segsum(ent_e, srcc, dstc)
        h1, Fv = _rgcn_layer1(cntB, gp1, n_rel, ent_e, rgcn_wmsg[0],
                              rgcn_wself[0], row2(rgcn_b[0]))
        gp2 = _sc_segsum(h1, srcc, dstc)
        ent_e = _rgcn_layer2(cntB, gp2, Fv, h1, rgcn_wmsg[1], rgcn_wself[1],
                             row2(rgcn_b[1]), ent_e, lin_w, row2(lin_b))
        rel_e = n_rel

    idx3 = jnp.stack([subj, rel, obj]).astype(jnp.int32)
    idx3 = idx3.reshape(3, NW, B // NW).transpose(1, 0, 2)
    g3 = _sc_gather3(ent_e, rel_e, idx3)
    e_s, e_r, e_o = g3[0], g3[1], g3[2]

    obj_logit = _obj_logits(e_s, e_r, wf_obj, obj_conv_b.reshape(1, C),
                            fcp_obj, row2(obj_fc_b), ent_e)
    tk = _topk(obj_logit)

    tkf = tk.reshape(NW, B * TOPK // NW)
    tkf = jnp.pad(tkf, ((0, 0), (0, GK_PAD - B * TOPK // NW)))
    pred = _sc_gather_topk(ent_e, tkf.reshape(NW, GK_CH, 128))
    pred = pred.reshape(NW, GK_PAD, H)[:, :B * TOPK // NW]
    pred = pred.reshape(B * TOPK, H)

    obj_logit2 = _mha_logits(e_s, e_r, pred, lin2_w, row2(lin2_b), mha_inw,
                             row2(mha_inb), mha_outw, row2(mha_outb), ent_e)
    rel_logit = _rel_logits(e_s, e_o, wf_rel, rel_conv_b.reshape(1, C),
                            fcp_rel, row2(rel_fc_b), rel_e)
    return (obj_logit2, rel_logit)


# speedup vs baseline: 3.0697x; 1.0121x over previous
import functools

import jax
import jax.numpy as jnp
from jax import lax
from jax.experimental import pallas as pl
from jax.experimental.pallas import tpu as pltpu
from jax.experimental.pallas import tpu_sc as plsc

N = 10000; R = 200; H = 128; T = 3; E = 160000; B = 1024
TOPK = 50; C = 50; KS = 3; NH = 4; L = 2
DH = H // NH

NC, NS = 2, 16
NW = NC * NS

TAB = R * N
TABP = TAB + 128
CNT_W = 20480
CNT_G = 10
CNT_WB = 124928
SEG_CH = 40
GK_CH = 13
GK_PAD = GK_CH * 128

_mesh = plsc.VectorSubcoreMesh(core_axis_name="c", subcore_axis_name="s")
_f32 = jnp.float32


def _zero_vmem_2d(ref, rows):
    z16 = jnp.zeros((16,), _f32)
    for i in range(rows):
        for j in range(ref.shape[1] // 16):
            ref[i, pl.ds(j * 16, 16)] = z16


def _zero_vmem_1d(ref):
    z16 = jnp.zeros((16,), _f32)
    for i in range(ref.shape[0] // 16):
        ref[pl.ds(i * 16, 16)] = z16


@functools.partial(
    pl.kernel,
    out_type=tuple(jax.ShapeDtypeStruct((TAB,), _f32) for _ in range(2 * T)),
    mesh=_mesh,
    scratch_types=[
        pltpu.VMEM_SHARED((TABP,), _f32),
        pltpu.VMEM((16, 128), jnp.int32),
        pltpu.VMEM((128,), _f32),
    ],
)
def _sc_counts(idx_hbm, zer_hbm, oa0, ob0, oa1, ob1, oa2, ob2, tab, idx_v,
               ones_v):
    c = lax.axis_index("c")
    s = lax.axis_index("s")
    for j in range(8):
        ones_v[pl.ds(j * 16, 16)] = jnp.ones((16,), _f32)
    outs = ((oa0, ob0), (oa1, ob1), (oa2, ob2))
    for t in range(T):
        pltpu.sync_copy(zer_hbm.at[pl.ds(s * CNT_WB, CNT_WB)],
                        tab.at[pl.ds(s * CNT_WB, CNT_WB)])

        @pl.when(s == 0)
        def _():
            pltpu.sync_copy(zer_hbm.at[pl.ds(NS * CNT_WB, TABP - NS * CNT_WB)],
                            tab.at[pl.ds(NS * CNT_WB, TABP - NS * CNT_WB)])

        plsc.subcore_barrier()
        for g in range(CNT_G):
            pltpu.sync_copy(idx_hbm.at[t, c, s, g], idx_v)
            for j in range(16):
                pltpu.sync_copy(ones_v, tab.at[idx_v.at[j]], add=True)
        plsc.subcore_barrier()
        oa, ob = outs[t]

        def wb(dst):
            pltpu.sync_copy(tab.at[pl.ds(s * CNT_WB, CNT_WB)],
                            dst.at[pl.ds(s * CNT_WB, CNT_WB)])

            @pl.when(s == 0)
            def _():
                pltpu.sync_copy(tab.at[pl.ds(NS * CNT_WB, TAB - NS * CNT_WB)],
                                dst.at[pl.ds(NS * CNT_WB, TAB - NS * CNT_WB)])

        @pl.when(c == 0)
        def _():
            wb(oa)

        @pl.when(c == 1)
        def _():
            wb(ob)

        plsc.subcore_barrier()


@functools.partial(
    pl.kernel,
    out_type=jax.ShapeDtypeStruct((2, N, H), _f32),
    mesh=_mesh,
    scratch_types=[
        pltpu.VMEM_SHARED((N + 16, H), _f32),
        pltpu.VMEM((SEG_CH, 128), jnp.int32),
        pltpu.VMEM((SEG_CH, 128), jnp.int32),
        pltpu.VMEM((128, H), _f32),
        pltpu.VMEM((128, H), _f32),
        pltpu.SemaphoreType.DMA,
        pltpu.SemaphoreType.DMA,
    ],
)
def _sc_segsum(h_hbm, src_hbm, dst_hbm, zer_hbm, out_hbm, acc, src_v, dst_v,
               rows0, rows1, sem0, sem1):
    c = lax.axis_index("c")
    s = lax.axis_index("s")
    wid = s * NC + c
    pltpu.sync_copy(src_hbm.at[wid], src_v)
    pltpu.sync_copy(dst_hbm.at[wid], dst_v)
    pltpu.sync_copy(zer_hbm.at[pl.ds(s * 624, 624), :],
                    acc.at[pl.ds(s * 624, 624), :])

    @pl.when(s == 0)
    def _():
        pltpu.sync_copy(zer_hbm.at[pl.ds(9984, 32), :],
                        acc.at[pl.ds(9984, 32), :])

    plsc.subcore_barrier()
    for j in range(SEG_CH):
        pltpu.async_copy(h_hbm.at[src_v.at[j]], rows0, sem0).wait()
        pltpu.sync_copy(rows0, acc.at[dst_v.at[j]], add=True)
    plsc.subcore_barrier()
    pltpu.sync_copy(acc.at[pl.ds(s * 624, 624), :],
                    out_hbm.at[c, pl.ds(s * 624, 624), :])

    @pl.when(s == NS - 1)
    def _():
        pltpu.sync_copy(acc.at[pl.ds(9984, 16), :],
                        out_hbm.at[c, pl.ds(9984, 16), :])


@functools.partial(
    pl.kernel,
    out_type=jax.ShapeDtypeStruct((3, B, H), _f32),
    mesh=_mesh,
    scratch_types=[
        pltpu.VMEM((3, 32), jnp.int32),
        pltpu.VMEM((32, H), _f32),
        pltpu.SemaphoreType.DMA,
    ],
)
def _sc_gather3(ent_hbm, rel_hbm, idx_hbm, out_hbm, idx_v, rows_v, sem):
    c = lax.axis_index("c")
    s = lax.axis_index("s")
    wid = s * NC + c
    pltpu.sync_copy(idx_hbm.at[wid], idx_v)
    for k, tbl in ((0, ent_hbm), (1, rel_hbm), (2, ent_hbm)):
        pltpu.async_copy(tbl.at[idx_v.at[k]], rows_v, sem).wait()
        pltpu.sync_copy(rows_v, out_hbm.at[k, pl.ds(wid * 32, 32), :])


@functools.partial(
    pl.kernel,
    out_type=jax.ShapeDtypeStruct((NW * GK_PAD, H), _f32),
    mesh=_mesh,
    scratch_types=[
        pltpu.VMEM((GK_CH, 128), jnp.int32),
        pltpu.VMEM((128, H), _f32),
        pltpu.SemaphoreType.DMA,
    ],
)
def _sc_gather_topk(ent_hbm, idx_hbm, out_hbm, idx_v, rows_v, sem):
    c = lax.axis_index("c")
    s = lax.axis_index("s")
    wid = s * NC + c
    pltpu.sync_copy(idx_hbm.at[wid], idx_v)
    for j in range(GK_CH):
        pltpu.async_copy(ent_hbm.at[idx_v.at[j]], rows_v, sem).wait()
        pltpu.sync_copy(rows_v, out_hbm.at[pl.ds(wid * GK_PAD + j * 128, 128), :])


def _dot(a, b):
    return lax.dot_general(a, b, (((1,), (0,)), ((), ())),
                           preferred_element_type=_f32)


def _dott(a, b):
    return lax.dot_general(a, b, (((1,), (1,)), ((), ())),
                           preferred_element_type=_f32)


def _l2n(x):
    n = jnp.sqrt(jnp.sum(x * x, axis=1, keepdims=True))
    return x / jnp.maximum(n, 1e-12)


def _l2norm_body(x_ref, o_ref):
    o_ref[...] = _l2n(x_ref[...])


def _l2norm(x, blk):
    m = x.shape[0]
    return pl.pallas_call(
        _l2norm_body,
        grid=(m // blk,),
        in_specs=[pl.BlockSpec((blk, H), lambda i: (i, 0))],
        out_specs=pl.BlockSpec((blk, H), lambda i: (i, 0)),
        out_shape=jax.ShapeDtypeStruct((m, H), _f32),
    )(x)


NP = 10240
NCH = 5
NBLK = NP // NCH


def _rel_body(cntA_ref, ent_ref, rel_emb_ref, rel_e_ref, wih_ref, whh_ref,
              bih_ref, bhh_ref, nrel_ref):
    m = (cntA_ref[...] > 0).astype(_f32)
    s = _dot(m, ent_ref[...])
    cnt = jnp.sum(m, axis=1, keepdims=True)
    rel_ent = jnp.where(cnt > 0, s / cnt, 0.0)
    x = jnp.concatenate([rel_ent, rel_emb_ref[...]], axis=1)
    h = rel_e_ref[...]
    gi = _dott(x, wih_ref[...]) + bih_ref[...]
    gh = _dott(h, whh_ref[...]) + bhh_ref[...]
    r = jax.nn.sigmoid(gi[:, :H] + gh[:, :H])
    z = jax.nn.sigmoid(gi[:, H:2 * H] + gh[:, H:2 * H])
    n = jnp.tanh(gi[:, 2 * H:] + r * gh[:, 2 * H:])
    nrel_ref[...] = _l2n((1.0 - z) * n + z * h)


def _rel_update(cntA, ent_e, rel_embeds, rel_e, wih, whh, bih, bhh):
    return pl.pallas_call(
        _rel_body,
        out_shape=jax.ShapeDtypeStruct((R, H), _f32),
    )(cntA, ent_e, rel_embeds, rel_e, wih, whh, bih, bhh)


LBLK = 1000


def _layer1_body(cnt_ref, gp_ref, nrel_ref, h_ref, wmsg_ref, wself_ref, b_ref,
                 h1_ref, F_ref):
    Fv = _dot(cnt_ref[...], nrel_ref[...])
    deg = jnp.maximum(jnp.sum(cnt_ref[...], axis=1, keepdims=True), 1.0)
    G = gp_ref[0] + gp_ref[1]
    agg = _dot(G + Fv, wmsg_ref[...]) / deg
    h1_ref[...] = jax.nn.relu(agg + _dot(h_ref[...], wself_ref[...]) +
                              b_ref[...])
    F_ref[...] = Fv


def _layer2_body(cnt_ref, gp_ref, F_ref, h_ref, wmsg_ref, wself_ref, b_ref,
                 ent_ref, linw_ref, linb_ref, out_ref):
    deg = jnp.maximum(jnp.sum(cnt_ref[...], axis=1, keepdims=True), 1.0)
    G = gp_ref[0] + gp_ref[1]
    agg = _dot(G + F_ref[...], wmsg_ref[...]) / deg
    hh = jax.nn.relu(agg + _dot(h_ref[...], wself_ref[...]) + b_ref[...])
    hn = _l2n(hh)
    e = ent_ref[...]
    u = jax.nn.sigmoid(_dott(e, linw_ref[...]) + linb_ref[...])
    out_ref[...] = e + u * (hn - e)


def _rgcn_layer1(cntB, gp, nrel, h, wmsg, wself, b):
    blk = lambda r, c, f=None: pl.BlockSpec((r, c), f or (lambda i: (0, 0)))
    return pl.pallas_call(
        _layer1_body,
        grid=(N // LBLK,),
        in_specs=[
            pl.BlockSpec((LBLK, R), lambda i: (i, 0)),
            pl.BlockSpec((2, LBLK, H), lambda i: (0, i, 0)),
            blk(R, H),
            pl.BlockSpec((LBLK, H), lambda i: (i, 0)),
            blk(H, H), blk(H, H), blk(1, H),
        ],
        out_specs=[pl.BlockSpec((LBLK, H), lambda i: (i, 0)),
                   pl.BlockSpec((LBLK, H), lambda i: (i, 0))],
        out_shape=[jax.ShapeDtypeStruct((N, H), _f32),
                   jax.ShapeDtypeStruct((N, H), _f32)],
    )(cntB, gp, nrel, h, wmsg, wself, b)


def _rgcn_layer2(cntB, gp, F, h, wmsg, wself, b, ent_e, lin_w, lin_b):
    blk = lambda r, c: pl.BlockSpec((r, c), lambda i: (0, 0))
    return pl.pallas_call(
        _layer2_body,
        grid=(N // LBLK,),
        in_specs=[
            pl.BlockSpec((LBLK, R), lambda i: (i, 0)),
            pl.BlockSpec((2, LBLK, H), lambda i: (0, i, 0)),
            pl.BlockSpec((LBLK, H), lambda i: (i, 0)),
            pl.BlockSpec((LBLK, H), lambda i: (i, 0)),
            blk(H, H), blk(H, H), blk(1, H),
            pl.BlockSpec((LBLK, H), lambda i: (i, 0)),
            blk(H, H), blk(1, H),
        ],
        out_specs=pl.BlockSpec((LBLK, H), lambda i: (i, 0)),
        out_shape=jax.ShapeDtypeStruct((N, H), _f32),
    )(cntB, gp, F, h, wmsg, wself, b, ent_e, lin_w, lin_b)


def _conv_transe_q(a, bb, wf_ref, cb_ref, fcp_ref, fcb_ref, rows):
    xs = (
        jnp.concatenate([jnp.zeros((rows, 1), _f32), a[:, :-1]], axis=1),
        a,
        jnp.concatenate([a[:, 1:], jnp.zeros((rows, 1), _f32)], axis=1),
        jnp.concatenate([jnp.zeros((rows, 1), _f32), bb[:, :-1]], axis=1),
        bb,
        jnp.concatenate([bb[:, 1:], jnp.zeros((rows, 1), _f32)], axis=1),
    )
    y = jnp.zeros((rows, H, C), _f32)
    for t in range(6):
        y = y + xs[t][:, :, None] * wf_ref[t:t + 1, :][None, :, :]
    y = jax.nn.relu(y + cb_ref[...][None])
    return jax.nn.relu(_dott(y.reshape(rows, H * C), fcp_ref[...]) +
                       fcb_ref[...])


OBLK = 128


def _objlogit_body(es_ref, er_ref, wf_ref, cb_ref, fcp_ref, fcb_ref, ent_ref,
                   out_ref, q_s):
    j = pl.program_id(1)

    @pl.when(j == 0)
    def _():
        q_s[...] = _conv_transe_q(es_ref[...], er_ref[...], wf_ref, cb_ref,
                                  fcp_ref, fcb_ref, OBLK)

    col = j * NBLK + lax.broadcasted_iota(jnp.int32, (1, NBLK), 1)
    out_ref[...] = jnp.where(col < N, _dott(q_s[...], ent_ref[...]),
                             -jnp.inf)


def _obj_logits(es, er, wf, cb, fcp, fcb, ent_e):
    blk = lambda r, c: pl.BlockSpec((r, c), lambda i, j: (0, 0))
    return pl.pallas_call(
        _objlogit_body,
        grid=(B // OBLK, NCH),
        in_specs=[
            pl.BlockSpec((OBLK, H), lambda i, j: (i, 0)),
            pl.BlockSpec((OBLK, H), lambda i, j: (i, 0)),
            blk(6, C), blk(1, C), blk(H, H * C), blk(1, H),
            pl.BlockSpec((NBLK, H), lambda i, j: (j, 0)),
        ],
        out_specs=pl.BlockSpec((OBLK, NBLK), lambda i, j: (i, j)),
        out_shape=jax.ShapeDtypeStruct((B, NP), _f32),
        scratch_shapes=[pltpu.VMEM((OBLK, H), _f32)],
    )(es, er, wf, cb, fcp, fcb, ent_e)


def _topk_body(lg_ref, oi_ref, x_s):
    x_s[...] = lg_ref[...]
    cols = lax.broadcasted_iota(jnp.int32, (OBLK, NP), 1)
    colsk = lax.broadcasted_iota(jnp.int32, (OBLK, TOPK), 1)

    def it(k, acc):
        v = x_s[...]
        m = jnp.max(v, axis=1, keepdims=True)
        am = jnp.min(jnp.where(v >= m, cols, NP), axis=1, keepdims=True)
        x_s[...] = jnp.where(cols == am, -jnp.inf, v)
        return jnp.where(colsk == k, am, acc)

    oi_ref[...] = lax.fori_loop(0, TOPK, it,
                                jnp.zeros((OBLK, TOPK), jnp.int32))


def _topk(logits):
    return pl.pallas_call(
        _topk_body,
        grid=(B // OBLK,),
        in_specs=[pl.BlockSpec((OBLK, NP), lambda i: (i, 0))],
        out_specs=pl.BlockSpec((OBLK, TOPK), lambda i: (i, 0)),
        out_shape=jax.ShapeDtypeStruct((B, TOPK), jnp.int32),
        scratch_shapes=[pltpu.VMEM((OBLK, NP), _f32)],
    )(logits)


MBLK = 128


def _mha_body(es_ref, er_ref, pred_ref, l2w_ref, l2b_ref, inw_ref, inb_ref,
              outw_ref, outb_ref, ent_ref, out_ref, rr_s):
    @pl.when(pl.program_id(1) == 0)
    def _():
        x2 = jnp.concatenate([es_ref[...], er_ref[...]], axis=1)
        q2 = _dott(x2, l2w_ref[...]) + l2b_ref[...]
        kv = pred_ref[...]
        Q = _dott(q2, inw_ref[0:H]) + inb_ref[:, 0:H]
        Kk = _dott(kv, inw_ref[H:2 * H]) + inb_ref[:, H:2 * H]
        V = _dott(kv, inw_ref[2 * H:]) + inb_ref[:, 2 * H:]
        Q4 = Q.reshape(MBLK, 1, NH, DH)
        K4 = Kk.reshape(MBLK, TOPK, NH, DH)
        V4 = V.reshape(MBLK, TOPK, NH, DH)
        sc = jnp.sum(Q4 * K4, axis=3) / jnp.sqrt(jnp.float32(DH))
        sc = sc - jnp.max(sc, axis=1, keepdims=True)
        ex = jnp.exp(sc)
        att = ex / jnp.sum(ex, axis=1, keepdims=True)
        o = jnp.sum(att[:, :, :, None] * V4, axis=1).reshape(MBLK, H)
        rr_s[...] = _dott(o, outw_ref[...]) + outb_ref[...]

    out_ref[...] = _dott(rr_s[...], ent_ref[...])


def _mha_logits(es, er, pred, l2w, l2b, inw, inb, outw, outb, ent_e):
    blk = lambda r, c: pl.BlockSpec((r, c), lambda i, j: (0, 0))
    return pl.pallas_call(
        _mha_body,
        grid=(B // MBLK, NCH),
        in_specs=[
            pl.BlockSpec((MBLK, H), lambda i, j: (i, 0)),
            pl.BlockSpec((MBLK, H), lambda i, j: (i, 0)),
            pl.BlockSpec((MBLK * TOPK, H), lambda i, j: (i, 0)),
            blk(H, 2 * H), blk(1, H), blk(3 * H, H), blk(1, 3 * H),
            blk(H, H), blk(1, H),
            pl.BlockSpec((NBLK, H), lambda i, j: (j, 0)),
        ],
        out_specs=pl.BlockSpec((MBLK, NBLK), lambda i, j: (i, j)),
        out_shape=jax.ShapeDtypeStruct((B, N), _f32),
        scratch_shapes=[pltpu.VMEM((MBLK, H), _f32)],
    )(es, er, pred, l2w, l2b, inw, inb, outw, outb, ent_e)


def _rellogit_body(es_ref, eo_ref, wf_ref, cb_ref, fcp_ref, fcb_ref, rel_ref,
                   out_ref):
    q = _conv_transe_q(es_ref[...], eo_ref[...], wf_ref, cb_ref, fcp_ref,
                       fcb_ref, OBLK)
    out_ref[...] = _dott(q, rel_ref[...])


def _rel_logits(es, eo, wf, cb, fcp, fcb, rel_e):
    blk = lambda r, c: pl.BlockSpec((r, c), lambda i: (0, 0))
    return pl.pallas_call(
        _rellogit_body,
        grid=(B // OBLK,),
        in_specs=[
            pl.BlockSpec((OBLK, H), lambda i: (i, 0)),
            pl.BlockSpec((OBLK, H), lambda i: (i, 0)),
            blk(6, C), blk(1, C), blk(H, H * C), blk(1, H),
            blk(R, H),
        ],
        out_specs=pl.BlockSpec((OBLK, R), lambda i: (i, 0)),
        out_shape=jax.ShapeDtypeStruct((B, R), _f32),
    )(es, eo, wf, cb, fcp, fcb, rel_e)


def _pad_chunks(flat, per_tile, nchunks, pad_val):
    x = flat.reshape(-1, per_tile)
    pad = nchunks * 128 - per_tile
    x = jnp.pad(x, ((0, 0), (0, pad)), constant_values=pad_val)
    return x.reshape(-1, nchunks, 128).astype(jnp.int32)


def kernel(edge_src, edge_dst, edge_rid, subj, rel, obj, ent_embeds,
           rel_embeds, lin_w, lin_b, gru_wih, gru_whh, gru_bih, gru_bhh,
           rgcn_wmsg, rgcn_wself, rgcn_b, obj_conv_w, obj_conv_b, obj_fc_w,
           obj_fc_b, rel_conv_w, rel_conv_b, rel_fc_w, rel_fc_b, lin2_w,
           lin2_b, mha_inw, mha_inb, mha_outw, mha_outb):
    f32 = _f32
    edge_src = edge_src.astype(jnp.int32)
    edge_dst = edge_dst.astype(jnp.int32)
    edge_rid = edge_rid.astype(jnp.int32)

    row2 = lambda v: v.reshape(1, -1).astype(f32)
    wf_obj = obj_conv_w.transpose(1, 2, 0).reshape(6, C).astype(f32)
    fcp_obj = obj_fc_w.reshape(H, C, H).transpose(0, 2, 1).reshape(H, H * C)
    wf_rel = rel_conv_w.transpose(1, 2, 0).reshape(6, C).astype(f32)
    fcp_rel = rel_fc_w.reshape(H, C, H).transpose(0, 2, 1).reshape(H, H * C)

    ent_e = _l2norm(ent_embeds.astype(f32), NBLK)
    rel_e = _l2norm(rel_embeds.astype(f32), R)

    zer1 = jnp.zeros((TABP,), f32)
    zer2 = jnp.zeros((N + 16, H), f32)

    idx_ts = []
    for t in range(T):
        src, dst, rid = edge_src[t], edge_dst[t], edge_rid[t]
        idxA = _pad_chunks(jnp.concatenate([rid * N + src, rid * N + dst]),
                           2 * E // NS, CNT_W // 128, TAB)
        idxB = _pad_chunks(dst * R + rid, E // NS, CNT_W // 128, TAB)
        idx_ts.append(jnp.stack([idxA, idxB]))
    idx_cnt = jnp.stack(idx_ts).reshape(T, 2, NS, CNT_G, 16, 128)
    cnts = _sc_counts(idx_cnt, zer1)

    for t in range(T):
        src, dst = edge_src[t], edge_dst[t]
        cntA = cnts[2 * t].reshape(R, N)
        cntB = cnts[2 * t + 1].reshape(N, R)

        n_rel = _rel_update(cntA, ent_e, rel_embeds.astype(f32), rel_e,
                            gru_wih, gru_whh, row2(gru_bih), row2(gru_bhh))

        srcc = _pad_chunks(src, E // NW, SEG_CH, 0)
        dstc = _pad_chunks(dst, E // NW, SEG_CH, N)

        gp1 = _sc_segsum(ent_e, srcc, dstc, zer2)
        h1, Fv = _rgcn_layer1(cntB, gp1, n_rel, ent_e, rgcn_wmsg[0],
                              rgcn_wself[0], row2(rgcn_b[0]))
        gp2 = _sc_segsum(h1, srcc, dstc, zer2)
        ent_e = _rgcn_layer2(cntB, gp2, Fv, h1, rgcn_wmsg[1], rgcn_wself[1],
                             row2(rgcn_b[1]), ent_e, lin_w, row2(lin_b))
        rel_e = n_rel

    idx3 = jnp.stack([subj, rel, obj]).astype(jnp.int32)
    idx3 = idx3.reshape(3, NW, B // NW).transpose(1, 0, 2)
    g3 = _sc_gather3(ent_e, rel_e, idx3)
    e_s, e_r, e_o = g3[0], g3[1], g3[2]

    obj_logit = _obj_logits(e_s, e_r, wf_obj, obj_conv_b.reshape(1, C),
                            fcp_obj, row2(obj_fc_b), ent_e)
    tk = _topk(obj_logit)

    tkf = tk.reshape(NW, B * TOPK // NW)
    tkf = jnp.pad(tkf, ((0, 0), (0, GK_PAD - B * TOPK // NW)))
    pred = _sc_gather_topk(ent_e, tkf.reshape(NW, GK_CH, 128))
    pred = pred.reshape(NW, GK_PAD, H)[:, :B * TOPK // NW]
    pred = pred.reshape(B * TOPK, H)

    obj_logit2 = _mha_logits(e_s, e_r, pred, lin2_w, row2(lin2_b), mha_inw,
                             row2(mha_inb), mha_outw, row2(mha_outb), ent_e)
    rel_logit = _rel_logits(e_s, e_o, wf_rel, rel_conv_b.reshape(1, C),
                            fcp_rel, row2(rel_fc_b), rel_e)
    return (obj_logit2, rel_logit)
